# baseline (speedup 1.0000x reference)
import functools

import jax
import jax.numpy as jnp
from jax.experimental import pallas as pl
from jax.experimental.pallas import tpu as pltpu

_MARGIN = 1.0
_LEVEL = 131072.0
_KEY_CO = 512.0
_SBJ_CO_X = 1024.0
_SBJ_CO_Y = -128.0
_AUG = 128
_NCHUNK = 8


def _triplet_kernel(emb_ref, lbl_ref, sbj_ref, out_ref, ycat_ref):
    y = emb_ref[...]
    b, d = y.shape
    sbj = sbj_ref[0, :]
    key = sbj * 4 + lbl_ref[0, :]

    sqy = jnp.sum(y * y, axis=1)
    c = jax.lax.broadcasted_iota(jnp.int32, (b, _AUG), 1)
    key_b = key[:, None]
    sbj_b = sbj[:, None]

    sqy_hi = sqy.astype(jnp.bfloat16).astype(jnp.float32)
    sqy_lo = sqy - sqy_hi
    yaug = (jnp.where(c == key_b, _KEY_CO, 0.0)
            + jnp.where(c == 32 + sbj_b, _SBJ_CO_Y, 0.0)
            + jnp.where(c == 40, sqy_hi[:, None], 0.0)
            + jnp.where(c == 41, sqy_lo[:, None], 0.0))
    ycat_ref[:, :d] = y.astype(jnp.bfloat16)
    ycat_ref[:, d:] = yaug.astype(jnp.bfloat16)

    xaug = (jnp.where(c == key_b, _KEY_CO, 0.0)
            + jnp.where(c == 32 + sbj_b, _SBJ_CO_X, 0.0)
            + jnp.where((c == 40) | (c == 41), 1.0, 0.0))
    xcat = jnp.concatenate(
        [(-2.0 * y).astype(jnp.bfloat16), xaug.astype(jnp.bfloat16)], axis=1)

    dn = (((1,), (1,)), ((), ()))
    nb = b // _NCHUNK
    row_max = [None] * _NCHUNK
    row_min = [None] * _NCHUNK
    col_max = [None] * _NCHUNK
    col_min = [None] * _NCHUNK

    def _acc(cur, val, op):
        return val if cur is None else op(cur, val)

    for bi_ in range(_NCHUNK):
        xrow = xcat[bi_ * nb:(bi_ + 1) * nb, :]
        sq_i = sqy[bi_ * nb:(bi_ + 1) * nb]
        for bj in range(bi_, _NCHUNK):
            g = jax.lax.dot_general(xrow, ycat_ref[bj * nb:(bj + 1) * nb, :],
                                    dn, preferred_element_type=jnp.float32)
            parts = [g[:, j * 128:(j + 1) * 128] for j in range(nb // 128)]
            row_max[bi_] = _acc(row_max[bi_],
                                functools.reduce(jnp.maximum, parts),
                                jnp.maximum)
            row_min[bi_] = _acc(row_min[bi_],
                                functools.reduce(jnp.minimum, parts),
                                jnp.minimum)
            if bj > bi_:
                a = g + sq_i[:, None]
                col_max[bj] = _acc(col_max[bj], jnp.max(a, axis=0),
                                   jnp.maximum)
                col_min[bj] = _acc(col_min[bj], jnp.min(a, axis=0),
                                   jnp.minimum)

    reds_p = []
    reds_n = []
    for a_ in range(_NCHUNK):
        rp = jnp.max(row_max[a_], axis=1)
        rn = jnp.min(row_min[a_], axis=1)
        if col_max[a_] is not None:
            sq_j = sqy[a_ * nb:(a_ + 1) * nb]
            rp = jnp.maximum(rp, col_max[a_] - sq_j)
            rn = jnp.minimum(rn, col_min[a_] - sq_j)
        reds_p.append(rp)
        reds_n.append(rn)
    red_p = jnp.concatenate(reds_p)
    red_n = jnp.concatenate(reds_n)

    d_ap = jnp.sqrt(jnp.maximum(red_p - _LEVEL + sqy, 0.0))
    d_an = jnp.sqrt(jnp.maximum(red_n + _LEVEL + sqy, 0.0))

    kc = jax.lax.broadcasted_iota(jnp.int32, (32, b), 0)
    onehot = jnp.where(kc == key[None, :], 1.0, 0.0)
    hist = jnp.sum(onehot, axis=1, keepdims=True)
    cnt = jnp.sum(onehot * hist, axis=0)

    valid = (cnt > 1.5) & (red_n < -65536.0)
    per_anchor = jnp.maximum(d_ap - d_an + _MARGIN, 0.0)
    s = jnp.sum(jnp.where(valid, per_anchor, 0.0))
    cnt_v = jnp.sum(valid.astype(jnp.float32))
    loss = jnp.where(cnt_v > 0.0, s / jnp.maximum(cnt_v, 1.0), 0.0)
    out_ref[...] = jnp.full((1, 1), loss, dtype=jnp.float32)


def kernel(emb, labels, sbj):
    b, d = emb.shape
    lbl2 = labels.astype(jnp.int32).reshape(1, b)
    sbj2 = sbj.astype(jnp.int32).reshape(1, b)

    out = pl.pallas_call(
        _triplet_kernel,
        out_shape=jax.ShapeDtypeStruct((1, 1), jnp.float32),
        scratch_shapes=[
            pltpu.VMEM((b, d + _AUG), jnp.bfloat16),
        ],
    )(emb, lbl2, sbj2)
    return out.reshape(())


# speedup vs baseline: 1.0832x; 1.0832x over previous
import functools

import jax
import jax.numpy as jnp
from jax.experimental import pallas as pl
from jax.experimental.pallas import tpu as pltpu

_MARGIN = 1.0
_LEVEL = 131072.0
_KEY_CO = 512.0
_SBJ_CO_X = 1024.0
_SBJ_CO_Y = -128.0
_AUG = 128
_NCHUNK = 4


def _triplet_kernel(emb_ref, lbl_ref, sbj_ref, out_ref, ycat_ref):
    y = emb_ref[...]
    b, d = y.shape
    sbj = sbj_ref[0, :]
    key = sbj * 4 + lbl_ref[0, :]

    sqy = jnp.sum(y * y, axis=1)
    c = jax.lax.broadcasted_iota(jnp.int32, (b, _AUG), 1)
    key_b = key[:, None]
    sbj_b = sbj[:, None]

    sqy_hi = sqy.astype(jnp.bfloat16).astype(jnp.float32)
    sqy_lo = sqy - sqy_hi
    yaug = (jnp.where(c == key_b, _KEY_CO, 0.0)
            + jnp.where(c == 32 + sbj_b, _SBJ_CO_Y, 0.0)
            + jnp.where(c == 40, sqy_hi[:, None], 0.0)
            + jnp.where(c == 41, sqy_lo[:, None], 0.0))
    ycat_ref[:, :d] = y.astype(jnp.bfloat16)
    ycat_ref[:, d:] = yaug.astype(jnp.bfloat16)

    xaug = (jnp.where(c == key_b, _KEY_CO, 0.0)
            + jnp.where(c == 32 + sbj_b, _SBJ_CO_X, 0.0)
            + jnp.where((c == 40) | (c == 41), 1.0, 0.0))
    xcat = jnp.concatenate(
        [(-2.0 * y).astype(jnp.bfloat16), xaug.astype(jnp.bfloat16)], axis=1)

    dn = (((1,), (1,)), ((), ()))
    nb = b // _NCHUNK
    row_max = [None] * _NCHUNK
    row_min = [None] * _NCHUNK
    col_max = [None] * _NCHUNK
    col_min = [None] * _NCHUNK

    def _acc(cur, val, op):
        return val if cur is None else op(cur, val)

    for bi_ in range(_NCHUNK):
        xrow = xcat[bi_ * nb:(bi_ + 1) * nb, :]
        sq_i = sqy[bi_ * nb:(bi_ + 1) * nb]
        for bj in range(bi_, _NCHUNK):
            g = jax.lax.dot_general(xrow, ycat_ref[bj * nb:(bj + 1) * nb, :],
                                    dn, preferred_element_type=jnp.float32)
            parts = [g[:, j * 128:(j + 1) * 128] for j in range(nb // 128)]
            row_max[bi_] = _acc(row_max[bi_],
                                functools.reduce(jnp.maximum, parts),
                                jnp.maximum)
            row_min[bi_] = _acc(row_min[bi_],
                                functools.reduce(jnp.minimum, parts),
                                jnp.minimum)
            if bj > bi_:
                a = g + sq_i[:, None]
                col_max[bj] = _acc(col_max[bj], jnp.max(a, axis=0),
                                   jnp.maximum)
                col_min[bj] = _acc(col_min[bj], jnp.min(a, axis=0),
                                   jnp.minimum)

    reds_p = []
    reds_n = []
    for a_ in range(_NCHUNK):
        rp = jnp.max(row_max[a_], axis=1)
        rn = jnp.min(row_min[a_], axis=1)
        if col_max[a_] is not None:
            sq_j = sqy[a_ * nb:(a_ + 1) * nb]
            rp = jnp.maximum(rp, col_max[a_] - sq_j)
            rn = jnp.minimum(rn, col_min[a_] - sq_j)
        reds_p.append(rp)
        reds_n.append(rn)
    red_p = jnp.concatenate(reds_p)
    red_n = jnp.concatenate(reds_n)

    d_ap = jnp.sqrt(jnp.maximum(red_p - _LEVEL + sqy, 0.0))
    d_an = jnp.sqrt(jnp.maximum(red_n + _LEVEL + sqy, 0.0))

    kc = jax.lax.broadcasted_iota(jnp.int32, (32, b), 0)
    onehot = jnp.where(kc == key[None, :], 1.0, 0.0)
    hist = jnp.sum(onehot, axis=1, keepdims=True)
    cnt = jnp.sum(onehot * hist, axis=0)

    valid = (cnt > 1.5) & (red_n < -65536.0)
    per_anchor = jnp.maximum(d_ap - d_an + _MARGIN, 0.0)
    s = jnp.sum(jnp.where(valid, per_anchor, 0.0))
    cnt_v = jnp.sum(valid.astype(jnp.float32))
    loss = jnp.where(cnt_v > 0.0, s / jnp.maximum(cnt_v, 1.0), 0.0)
    out_ref[...] = jnp.full((1, 1), loss, dtype=jnp.float32)


def kernel(emb, labels, sbj):
    b, d = emb.shape
    lbl2 = labels.astype(jnp.int32).reshape(1, b)
    sbj2 = sbj.astype(jnp.int32).reshape(1, b)

    out = pl.pallas_call(
        _triplet_kernel,
        out_shape=jax.ShapeDtypeStruct((1, 1), jnp.float32),
        scratch_shapes=[
            pltpu.VMEM((b, d + _AUG), jnp.bfloat16),
        ],
    )(emb, lbl2, sbj2)
    return out.reshape(())
